# baseline (speedup 1.0000x reference)
import functools

import jax
import jax.numpy as jnp
from jax import lax
from jax.experimental import pallas as pl
from jax.experimental.pallas import tpu as pltpu
from jax.experimental.pallas import tpu_sc as plsc

GAMMA_C = 1.0
L2_C = 0.0001
NUM_EMBS = 2

NC = 2
NS = 16
LANES = 16
CHUNK = 128


def _sc_gather_kernel(B, D, D1, D2):
    nw = NC * NS
    per_w = B // nw
    nch = per_w // CHUNK
    f32 = jnp.float32
    i32 = jnp.int32
    mesh = plsc.VectorSubcoreMesh(
        core_axis_name="c", subcore_axis_name="s", num_cores=NC, num_subcores=NS
    )

    @functools.partial(
        pl.kernel,
        out_type=[
            jax.ShapeDtypeStruct((B, D), f32),
            jax.ShapeDtypeStruct((B, D1), f32),
            jax.ShapeDtypeStruct((B, D2), f32),
            jax.ShapeDtypeStruct((B, LANES), f32),
        ],
        mesh=mesh,
        compiler_params=pltpu.CompilerParams(use_tc_tiling_on_sc=False),
        scratch_types=[
            pltpu.VMEM((CHUNK,), i32),
            pltpu.VMEM((CHUNK,), i32),
            pltpu.VMEM((CHUNK,), i32),
            pltpu.VMEM((CHUNK,), i32),
            pltpu.VMEM((CHUNK,), i32),
            pltpu.VMEM((CHUNK,), i32),
            pltpu.VMEM((CHUNK,), i32),
            pltpu.VMEM((CHUNK, D), f32),
            pltpu.VMEM((CHUNK, D), f32),
            pltpu.VMEM((CHUNK, D), f32),
            pltpu.VMEM((CHUNK, D), f32),
            pltpu.VMEM((CHUNK, D), f32),
            pltpu.VMEM((CHUNK, D), f32),
            pltpu.VMEM((CHUNK, D), f32),
            pltpu.VMEM((CHUNK, D1), f32),
            pltpu.VMEM((CHUNK, D2), f32),
            pltpu.VMEM((CHUNK, LANES), f32),
            pltpu.SemaphoreType.DMA,
        ],
    )
    def sc_fn(meta, rel, e0t, e1t, ps, pr, po, nsb, nrb, nob, inpx,
              xg, e0g, e1g, dacc,
              ips, ipr, ipo, ins, inr, ino, iin,
              bsp, brp, bop, bsn, brn, bon, bx, be0, be1, bd, sem):
        wid = lax.axis_index("s") * NC + lax.axis_index("c")
        for c in range(nch):
            base = wid * per_w + c * CHUNK
            sl = pl.ds(base, CHUNK)
            pltpu.sync_copy(ps.at[sl], ips)
            pltpu.sync_copy(pr.at[sl], ipr)
            pltpu.sync_copy(po.at[sl], ipo)
            pltpu.sync_copy(nsb.at[sl], ins)
            pltpu.sync_copy(nrb.at[sl], inr)
            pltpu.sync_copy(nob.at[sl], ino)
            pltpu.sync_copy(inpx.at[sl], iin)
            cps = [
                pltpu.async_copy(meta.at[ips], bsp, sem),
                pltpu.async_copy(rel.at[ipr], brp, sem),
                pltpu.async_copy(meta.at[ipo], bop, sem),
                pltpu.async_copy(meta.at[ins], bsn, sem),
                pltpu.async_copy(rel.at[inr], brn, sem),
                pltpu.async_copy(meta.at[ino], bon, sem),
                pltpu.async_copy(meta.at[iin], bx, sem),
                pltpu.async_copy(e0t.at[iin], be0, sem),
                pltpu.async_copy(e1t.at[iin], be1, sem),
            ]
            for cp in cps:
                cp.wait()
            pltpu.sync_copy(bx, xg.at[sl])
            pltpu.sync_copy(be0, e0g.at[sl])
            pltpu.sync_copy(be1, e1g.at[sl])

            def row_body(r, carry):
                acc = jnp.zeros((LANES,), f32)
                for k in range(D // LANES):
                    ksl = pl.ds(k * LANES, LANES)
                    dp = bsp[r, ksl] + brp[r, ksl] - bop[r, ksl]
                    dn = bsn[r, ksl] + brn[r, ksl] - bon[r, ksl]
                    acc = acc + (dp * dp - dn * dn)
                bd[r, :] = acc
                return carry

            lax.fori_loop(0, CHUNK, row_body, 0)
            pltpu.sync_copy(bd, dacc.at[sl])

    return sc_fn


def _tc_combine_kernel(B, D, D1, D2, RB):
    f32 = jnp.float32
    G = B // RB

    def body(x_ref, e0_ref, e1_ref, w1_ref, w2_ref, dv_ref, w_ref, ms_ref,
             loss_ref, se_ref, sg_ref):
        i = pl.program_id(0)
        x = x_ref[...]
        d1 = jnp.dot(x, w1_ref[...], preferred_element_type=f32,
                     precision=lax.Precision.HIGHEST) - e0_ref[...]
        d2 = jnp.dot(x, w2_ref[...], preferred_element_type=f32,
                     precision=lax.Precision.HIGHEST) - e1_ref[...]
        s_emb = (jnp.sum(jnp.sqrt(jnp.sum(d1 * d1, axis=1))) +
                 jnp.sum(jnp.sqrt(jnp.sum(d2 * d2, axis=1))))
        sd = jnp.sum(dv_ref[...], axis=1)
        hinge = jnp.maximum(0.0, 1.0 + sd)
        wv = w_ref[0, 0, :]
        ms = ms_ref[0, 0]
        s_graph = ms * jnp.sum(hinge) + jnp.sum(wv * hinge)

        @pl.when(i == 0)
        def _():
            se_ref[0, 0] = 0.0
            sg_ref[0, 0] = 0.0

        se_ref[0, 0] += s_emb
        sg_ref[0, 0] += s_graph

        @pl.when(i == pl.num_programs(0) - 1)
        def _():
            wn = jnp.sum(jnp.abs(w1_ref[...])) + jnp.sum(jnp.abs(w2_ref[...]))
            se_t = se_ref[0, 0] + L2_C * wn
            sg_t = sg_ref[0, 0]
            se_ref[0, 0] = se_t
            loss_ref[0, 0] = se_t + (NUM_EMBS / 2.0) * GAMMA_C * sg_t / (ms + ms)

    return pl.pallas_call(
        body,
        grid=(G,),
        in_specs=[
            pl.BlockSpec((RB, D), lambda i: (i, 0)),
            pl.BlockSpec((RB, D1), lambda i: (i, 0)),
            pl.BlockSpec((RB, D2), lambda i: (i, 0)),
            pl.BlockSpec((D, D1), lambda i: (0, 0)),
            pl.BlockSpec((D, D2), lambda i: (0, 0)),
            pl.BlockSpec((RB, LANES), lambda i: (i, 0)),
            pl.BlockSpec((1, 1, RB), lambda i: (i, 0, 0)),
            pl.BlockSpec(memory_space=pltpu.SMEM),
        ],
        out_specs=[
            pl.BlockSpec(memory_space=pltpu.SMEM),
            pl.BlockSpec(memory_space=pltpu.SMEM),
            pl.BlockSpec(memory_space=pltpu.SMEM),
        ],
        out_shape=[
            jax.ShapeDtypeStruct((1, 1), f32),
            jax.ShapeDtypeStruct((1, 1), f32),
            jax.ShapeDtypeStruct((1, 1), f32),
        ],
    )


def kernel(inp, pos_samples, neg_samples, weight, meanscore, metaemb, relemb,
           emb0, emb1, W1, W2):
    B = inp.shape[0]
    D = metaemb.shape[1]
    D1 = emb0.shape[1]
    D2 = emb1.shape[1]
    i32 = jnp.int32

    inp_i = inp.astype(i32)
    ps = pos_samples[:, 0].astype(i32)
    pr = pos_samples[:, 1].astype(i32)
    po = pos_samples[:, 2].astype(i32)
    nsb = neg_samples[:, 0].astype(i32)
    nrb = neg_samples[:, 1].astype(i32)
    nob = neg_samples[:, 2].astype(i32)

    sc_fn = _sc_gather_kernel(B, D, D1, D2)
    xg, e0g, e1g, dacc = sc_fn(metaemb, relemb, emb0, emb1,
                               ps, pr, po, nsb, nrb, nob, inp_i)

    RB = 2048
    tc_fn = _tc_combine_kernel(B, D, D1, D2, RB)
    loss11, se11, sg11 = tc_fn(xg, e0g, e1g, W1, W2, dacc,
                               weight.reshape(B // RB, 1, RB),
                               meanscore.reshape(1, 1))
    return (loss11.reshape(1), se11.reshape(()), sg11.reshape(()))


# speedup vs baseline: 1.1009x; 1.1009x over previous
import functools

import jax
import jax.numpy as jnp
from jax import lax
from jax.experimental import pallas as pl
from jax.experimental.pallas import tpu as pltpu
from jax.experimental.pallas import tpu_sc as plsc

GAMMA_C = 1.0
L2_C = 0.0001
NUM_EMBS = 2

NC = 2
NS = 16
LANES = 16
CHUNK = 128
PK = 128


def _sc_gather_kernel(B, D):
    nw = NC * NS
    per_w = B // nw
    nch = per_w // CHUNK
    f32 = jnp.float32
    i32 = jnp.int32
    mesh = plsc.VectorSubcoreMesh(
        core_axis_name="c", subcore_axis_name="s", num_cores=NC, num_subcores=NS
    )

    @functools.partial(
        pl.kernel,
        out_type=[
            jax.ShapeDtypeStruct((B, PK), f32),
            jax.ShapeDtypeStruct((B, PK), f32),
        ],
        mesh=mesh,
        compiler_params=pltpu.CompilerParams(use_tc_tiling_on_sc=True),
        scratch_types=[
            pltpu.VMEM((CHUNK,), i32),
            pltpu.VMEM((CHUNK,), i32),
            pltpu.VMEM((CHUNK,), i32),
            pltpu.VMEM((CHUNK, PK), f32),
            pltpu.VMEM((CHUNK, PK), f32),
            pltpu.VMEM((CHUNK, PK), f32),
            pltpu.VMEM((CHUNK, PK), f32),
            pltpu.VMEM((CHUNK, PK), f32),
            pltpu.VMEM((CHUNK, LANES), f32),
            pltpu.SemaphoreType.DMA,
        ],
    )
    def sc_fn(ta, tb, ps2, pr2, po2, ns2, nr2, no2, in2,
              out_a, out_b,
              ia, ib, ic, g1, g2, g3, bufA, bufB, bd, sem):
        wid = lax.axis_index("s") * NC + lax.axis_index("c")
        for c in range(nch):
            g = wid * nch + c
            base = g * CHUNK
            rsl = pl.ds(base, CHUNK)

            pltpu.sync_copy(ps2.at[g], ia)
            pltpu.sync_copy(po2.at[g], ib)
            pltpu.sync_copy(pr2.at[g], ic)
            cps = [
                pltpu.async_copy(ta.at[ia], g1, sem),
                pltpu.async_copy(ta.at[ib], g2, sem),
                pltpu.async_copy(tb.at[ic], g3, sem),
            ]
            for cp in cps:
                cp.wait()

            def pos_body(r, carry):
                acc = jnp.zeros((LANES,), f32)
                for k in range(D // LANES):
                    ksl = pl.ds(k * LANES, LANES)
                    dp = g1[r, ksl] + g3[r, ksl] - g2[r, ksl]
                    acc = acc + dp * dp
                bd[r, :] = acc
                return carry

            lax.fori_loop(0, CHUNK, pos_body, 0)

            pltpu.sync_copy(ns2.at[g], ia)
            pltpu.sync_copy(no2.at[g], ib)
            pltpu.sync_copy(nr2.at[g], ic)
            cps = [
                pltpu.async_copy(ta.at[ia], g1, sem),
                pltpu.async_copy(ta.at[ib], g2, sem),
                pltpu.async_copy(tb.at[ic], g3, sem),
            ]
            for cp in cps:
                cp.wait()

            def neg_body(r, carry):
                acc = jnp.zeros((LANES,), f32)
                for k in range(D // LANES):
                    ksl = pl.ds(k * LANES, LANES)
                    dn = g1[r, ksl] + g3[r, ksl] - g2[r, ksl]
                    acc = acc + dn * dn
                bd[r, :] = bd[r, :] - acc
                return carry

            lax.fori_loop(0, CHUNK, neg_body, 0)

            pltpu.sync_copy(in2.at[g], ia)
            cps = [
                pltpu.async_copy(ta.at[ia], bufA, sem),
                pltpu.async_copy(tb.at[ia], bufB, sem),
            ]
            for cp in cps:
                cp.wait()
            pltpu.sync_copy(bufA, out_a.at[rsl])

            def pack_body(r, carry):
                bufB[r, pl.ds(0, LANES)] = bd[r, :]
                return carry

            lax.fori_loop(0, CHUNK, pack_body, 0)
            pltpu.sync_copy(bufB, out_b.at[rsl])

    return sc_fn


def _tc_combine_kernel(B, D, D1, D2, RB):
    f32 = jnp.float32
    G = B // RB

    def body(a_ref, b_ref, w1_ref, w2_ref, w_ref, ms_ref,
             loss_ref, se_ref, sg_ref):
        i = pl.program_id(0)
        x = a_ref[:, :D]
        e0 = a_ref[:, D:D + D1]
        dv = b_ref[:, :LANES]
        e1 = b_ref[:, D:D + D2]
        d1 = jnp.dot(x, w1_ref[...], preferred_element_type=f32,
                     precision=lax.Precision.HIGHEST) - e0
        d2 = jnp.dot(x, w2_ref[...], preferred_element_type=f32,
                     precision=lax.Precision.HIGHEST) - e1
        s_emb = (jnp.sum(jnp.sqrt(jnp.sum(d1 * d1, axis=1))) +
                 jnp.sum(jnp.sqrt(jnp.sum(d2 * d2, axis=1))))
        sd = jnp.sum(dv, axis=1)
        hinge = jnp.maximum(0.0, 1.0 + sd)
        wv = w_ref[0, 0, :]
        ms = ms_ref[0, 0]
        s_graph = ms * jnp.sum(hinge) + jnp.sum(wv * hinge)

        @pl.when(i == 0)
        def _():
            se_ref[0, 0] = 0.0
            sg_ref[0, 0] = 0.0

        se_ref[0, 0] += s_emb
        sg_ref[0, 0] += s_graph

        @pl.when(i == pl.num_programs(0) - 1)
        def _():
            wn = jnp.sum(jnp.abs(w1_ref[...])) + jnp.sum(jnp.abs(w2_ref[...]))
            se_t = se_ref[0, 0] + L2_C * wn
            sg_t = sg_ref[0, 0]
            se_ref[0, 0] = se_t
            loss_ref[0, 0] = se_t + (NUM_EMBS / 2.0) * GAMMA_C * sg_t / (ms + ms)

    return pl.pallas_call(
        body,
        grid=(G,),
        in_specs=[
            pl.BlockSpec((RB, PK), lambda i: (i, 0)),
            pl.BlockSpec((RB, PK), lambda i: (i, 0)),
            pl.BlockSpec((D, D1), lambda i: (0, 0)),
            pl.BlockSpec((D, D2), lambda i: (0, 0)),
            pl.BlockSpec((1, 1, RB), lambda i: (i, 0, 0)),
            pl.BlockSpec(memory_space=pltpu.SMEM),
        ],
        out_specs=[
            pl.BlockSpec(memory_space=pltpu.SMEM),
            pl.BlockSpec(memory_space=pltpu.SMEM),
            pl.BlockSpec(memory_space=pltpu.SMEM),
        ],
        out_shape=[
            jax.ShapeDtypeStruct((1, 1), f32),
            jax.ShapeDtypeStruct((1, 1), f32),
            jax.ShapeDtypeStruct((1, 1), f32),
        ],
    )


def kernel(inp, pos_samples, neg_samples, weight, meanscore, metaemb, relemb,
           emb0, emb1, W1, W2):
    B = inp.shape[0]
    D = metaemb.shape[1]
    D1 = emb0.shape[1]
    D2 = emb1.shape[1]
    V1 = metaemb.shape[0]
    i32 = jnp.int32
    f32 = jnp.float32

    ta = jnp.concatenate([metaemb, emb0], axis=1)
    tb = jnp.concatenate([relemb, emb1,
                          jnp.zeros((V1, PK - D - D2), f32)], axis=1)

    nct = B // CHUNK
    in2 = inp.astype(i32).reshape(nct, CHUNK)
    ps2 = pos_samples[:, 0].astype(i32).reshape(nct, CHUNK)
    pr2 = pos_samples[:, 1].astype(i32).reshape(nct, CHUNK)
    po2 = pos_samples[:, 2].astype(i32).reshape(nct, CHUNK)
    ns2 = neg_samples[:, 0].astype(i32).reshape(nct, CHUNK)
    nr2 = neg_samples[:, 1].astype(i32).reshape(nct, CHUNK)
    no2 = neg_samples[:, 2].astype(i32).reshape(nct, CHUNK)

    sc_fn = _sc_gather_kernel(B, D)
    out_a, out_b = sc_fn(ta, tb, ps2, pr2, po2, ns2, nr2, no2, in2)

    RB = 2048
    tc_fn = _tc_combine_kernel(B, D, D1, D2, RB)
    loss11, se11, sg11 = tc_fn(out_a, out_b, W1, W2,
                               weight.reshape(B // RB, 1, RB),
                               meanscore.reshape(1, 1))
    return (loss11.reshape(1), se11.reshape(()), sg11.reshape(()))


# speedup vs baseline: 1.1796x; 1.0715x over previous
import functools

import jax
import jax.numpy as jnp
from jax import lax
from jax.experimental import pallas as pl
from jax.experimental.pallas import tpu as pltpu
from jax.experimental.pallas import tpu_sc as plsc

GAMMA_C = 1.0
L2_C = 0.0001
NUM_EMBS = 2

NC = 2
NS = 16
LANES = 16
CHUNK = 32
PK = 128


def _sc_gather_kernel(B, D):
    nw = NC * NS
    per_w = B // nw
    nch = per_w // CHUNK
    f32 = jnp.float32
    i32 = jnp.int32
    mesh = plsc.VectorSubcoreMesh(
        core_axis_name="c", subcore_axis_name="s", num_cores=NC, num_subcores=NS
    )

    idx_scratch = [pltpu.VMEM((nch, CHUNK), i32) for _ in range(7)]
    set_scratch = []
    for _ in range(2):
        set_scratch += [pltpu.VMEM((CHUNK, PK), f32) for _ in range(8)]

    @functools.partial(
        pl.kernel,
        out_type=[
            jax.ShapeDtypeStruct((B, PK), f32),
            jax.ShapeDtypeStruct((B, PK), f32),
        ],
        mesh=mesh,
        compiler_params=pltpu.CompilerParams(use_tc_tiling_on_sc=True),
        scratch_types=idx_scratch + set_scratch + [
            pltpu.SemaphoreType.DMA,
            pltpu.SemaphoreType.DMA,
            pltpu.SemaphoreType.DMA,
            pltpu.SemaphoreType.DMA,
        ],
    )
    def sc_fn(ta, tb, ps2, pr2, po2, ns2, nr2, no2, in2,
              out_a, out_b,
              xps, xpr, xpo, xns, xnr, xno, xin,
              p1a, p2a, p3a, n1a, n2a, n3a, bAa, bBa,
              p1b, p2b, p3b, n1b, n2b, n3b, bAb, bBb,
              sg0, sg1, sw0, sw1):
        wid = lax.axis_index("s") * NC + lax.axis_index("c")
        row0 = wid * nch
        for src, dst in ((ps2, xps), (pr2, xpr), (po2, xpo), (ns2, xns),
                         (nr2, xnr), (no2, xno), (in2, xin)):
            pltpu.sync_copy(src.at[pl.ds(row0, nch)], dst)

        sets = (
            (p1a, p2a, p3a, n1a, n2a, n3a, bAa, bBa, sg0, sw0),
            (p1b, p2b, p3b, n1b, n2b, n3b, bAb, bBb, sg1, sw1),
        )
        gathers = [None, None]
        writes = [None, None]

        def fire(c):
            p1, p2, p3, n1, n2, n3, bA, bB, sg, _ = sets[c % 2]
            gathers[c % 2] = [
                pltpu.async_copy(ta.at[xps.at[c]], p1, sg),
                pltpu.async_copy(ta.at[xpo.at[c]], p2, sg),
                pltpu.async_copy(tb.at[xpr.at[c]], p3, sg),
                pltpu.async_copy(ta.at[xns.at[c]], n1, sg),
                pltpu.async_copy(ta.at[xno.at[c]], n2, sg),
                pltpu.async_copy(tb.at[xnr.at[c]], n3, sg),
                pltpu.async_copy(ta.at[xin.at[c]], bA, sg),
                pltpu.async_copy(tb.at[xin.at[c]], bB, sg),
            ]

        def consume(c):
            p1, p2, p3, n1, n2, n3, bA, bB, _, sw = sets[c % 2]
            for cp in gathers[c % 2]:
                cp.wait()

            def row_body(r, carry):
                accp = jnp.zeros((LANES,), f32)
                accn = jnp.zeros((LANES,), f32)
                for k in range(D // LANES):
                    ksl = pl.ds(k * LANES, LANES)
                    dp = p1[r, ksl] + p3[r, ksl] - p2[r, ksl]
                    accp = accp + dp * dp
                    dn = n1[r, ksl] + n3[r, ksl] - n2[r, ksl]
                    accn = accn + dn * dn
                bB[r, pl.ds(0, LANES)] = accp - accn
                return carry

            lax.fori_loop(0, CHUNK, row_body, 0)
            base = (row0 + c) * CHUNK
            rsl = pl.ds(base, CHUNK)
            writes[c % 2] = [
                pltpu.async_copy(bA, out_a.at[rsl], sw),
                pltpu.async_copy(bB, out_b.at[rsl], sw),
            ]

        for c in range(nch):
            if c >= 2:
                for cp in writes[c % 2]:
                    cp.wait()
            fire(c)
            if c >= 1:
                consume(c - 1)
        consume(nch - 1)
        for s in range(2):
            for cp in writes[s]:
                cp.wait()

    return sc_fn


def _tc_combine_kernel(B, D, D1, D2, RB):
    f32 = jnp.float32
    G = B // RB

    def body(a_ref, b_ref, w1_ref, w2_ref, w_ref, ms_ref,
             loss_ref, se_ref, sg_ref):
        i = pl.program_id(0)
        x = a_ref[:, :D]
        e0 = a_ref[:, D:D + D1]
        dv = b_ref[:, :LANES]
        e1 = b_ref[:, D:D + D2]
        d1 = jnp.dot(x, w1_ref[...], preferred_element_type=f32,
                     precision=lax.Precision.HIGHEST) - e0
        d2 = jnp.dot(x, w2_ref[...], preferred_element_type=f32,
                     precision=lax.Precision.HIGHEST) - e1
        s_emb = (jnp.sum(jnp.sqrt(jnp.sum(d1 * d1, axis=1))) +
                 jnp.sum(jnp.sqrt(jnp.sum(d2 * d2, axis=1))))
        sd = jnp.sum(dv, axis=1)
        hinge = jnp.maximum(0.0, 1.0 + sd)
        wv = w_ref[0, 0, :]
        ms = ms_ref[0, 0]
        s_graph = ms * jnp.sum(hinge) + jnp.sum(wv * hinge)

        @pl.when(i == 0)
        def _():
            se_ref[0, 0] = 0.0
            sg_ref[0, 0] = 0.0

        se_ref[0, 0] += s_emb
        sg_ref[0, 0] += s_graph

        @pl.when(i == pl.num_programs(0) - 1)
        def _():
            wn = jnp.sum(jnp.abs(w1_ref[...])) + jnp.sum(jnp.abs(w2_ref[...]))
            se_t = se_ref[0, 0] + L2_C * wn
            sg_t = sg_ref[0, 0]
            se_ref[0, 0] = se_t
            loss_ref[0, 0] = se_t + (NUM_EMBS / 2.0) * GAMMA_C * sg_t / (ms + ms)

    return pl.pallas_call(
        body,
        grid=(G,),
        in_specs=[
            pl.BlockSpec((RB, PK), lambda i: (i, 0)),
            pl.BlockSpec((RB, PK), lambda i: (i, 0)),
            pl.BlockSpec((D, D1), lambda i: (0, 0)),
            pl.BlockSpec((D, D2), lambda i: (0, 0)),
            pl.BlockSpec((1, 1, RB), lambda i: (i, 0, 0)),
            pl.BlockSpec(memory_space=pltpu.SMEM),
        ],
        out_specs=[
            pl.BlockSpec(memory_space=pltpu.SMEM),
            pl.BlockSpec(memory_space=pltpu.SMEM),
            pl.BlockSpec(memory_space=pltpu.SMEM),
        ],
        out_shape=[
            jax.ShapeDtypeStruct((1, 1), f32),
            jax.ShapeDtypeStruct((1, 1), f32),
            jax.ShapeDtypeStruct((1, 1), f32),
        ],
    )


def kernel(inp, pos_samples, neg_samples, weight, meanscore, metaemb, relemb,
           emb0, emb1, W1, W2):
    B = inp.shape[0]
    D = metaemb.shape[1]
    D1 = emb0.shape[1]
    D2 = emb1.shape[1]
    V1 = metaemb.shape[0]
    i32 = jnp.int32
    f32 = jnp.float32

    ta = jnp.concatenate([metaemb, emb0], axis=1)
    tb = jnp.concatenate([relemb, emb1,
                          jnp.zeros((V1, PK - D - D2), f32)], axis=1)

    nct = B // CHUNK
    in2 = inp.astype(i32).reshape(nct, CHUNK)
    ps2 = pos_samples[:, 0].astype(i32).reshape(nct, CHUNK)
    pr2 = pos_samples[:, 1].astype(i32).reshape(nct, CHUNK)
    po2 = pos_samples[:, 2].astype(i32).reshape(nct, CHUNK)
    ns2 = neg_samples[:, 0].astype(i32).reshape(nct, CHUNK)
    nr2 = neg_samples[:, 1].astype(i32).reshape(nct, CHUNK)
    no2 = neg_samples[:, 2].astype(i32).reshape(nct, CHUNK)

    sc_fn = _sc_gather_kernel(B, D)
    out_a, out_b = sc_fn(ta, tb, ps2, pr2, po2, ns2, nr2, no2, in2)

    RB = 2048
    tc_fn = _tc_combine_kernel(B, D, D1, D2, RB)
    loss11, se11, sg11 = tc_fn(out_a, out_b, W1, W2,
                               weight.reshape(B // RB, 1, RB),
                               meanscore.reshape(1, 1))
    return (loss11.reshape(1), se11.reshape(()), sg11.reshape(()))


# speedup vs baseline: 1.2355x; 1.0473x over previous
import functools

import jax
import jax.numpy as jnp
from jax import lax
from jax.experimental import pallas as pl
from jax.experimental.pallas import tpu as pltpu
from jax.experimental.pallas import tpu_sc as plsc

GAMMA_C = 1.0
L2_C = 0.0001
NUM_EMBS = 2

NC = 2
NS = 16
LANES = 16
CHUNK = 32
PK = 128


def _sc_gather_kernel(B, D):
    nw = NC * NS
    per_w = B // nw
    nch = per_w // CHUNK
    f32 = jnp.float32
    i32 = jnp.int32
    mesh = plsc.VectorSubcoreMesh(
        core_axis_name="c", subcore_axis_name="s", num_cores=NC, num_subcores=NS
    )

    idx_scratch = [pltpu.VMEM((nch, CHUNK), i32) for _ in range(7)]
    set_scratch = []
    for _ in range(2):
        set_scratch += [pltpu.VMEM((CHUNK, PK), f32) for _ in range(8)]

    @functools.partial(
        pl.kernel,
        out_type=[
            jax.ShapeDtypeStruct((B, PK), f32),
            jax.ShapeDtypeStruct((B, PK), f32),
        ],
        mesh=mesh,
        compiler_params=pltpu.CompilerParams(use_tc_tiling_on_sc=True),
        scratch_types=idx_scratch + set_scratch + [
            pltpu.SemaphoreType.DMA,
            pltpu.SemaphoreType.DMA,
            pltpu.SemaphoreType.DMA,
            pltpu.SemaphoreType.DMA,
        ],
    )
    def sc_fn(ta, tb, ps2, pr2, po2, ns2, nr2, no2, in2,
              out_a, out_b,
              xps, xpr, xpo, xns, xnr, xno, xin,
              p1a, p2a, p3a, n1a, n2a, n3a, bAa, bBa,
              p1b, p2b, p3b, n1b, n2b, n3b, bAb, bBb,
              sg0, sg1, sw0, sw1):
        wid = lax.axis_index("s") * NC + lax.axis_index("c")
        row0 = wid * nch
        for src, dst in ((ps2, xps), (pr2, xpr), (po2, xpo), (ns2, xns),
                         (nr2, xnr), (no2, xno), (in2, xin)):
            pltpu.sync_copy(src.at[pl.ds(row0, nch)], dst)

        sets = (
            (p1a, p2a, p3a, n1a, n2a, n3a, bAa, bBa, sg0, sw0),
            (p1b, p2b, p3b, n1b, n2b, n3b, bAb, bBb, sg1, sw1),
        )
        gathers = [None, None]
        writes = [None, None]

        def fire(c):
            p1, p2, p3, n1, n2, n3, bA, bB, sg, _ = sets[c % 2]
            gathers[c % 2] = [
                pltpu.async_copy(ta.at[xps.at[c]], p1, sg),
                pltpu.async_copy(ta.at[xpo.at[c]], p2, sg),
                pltpu.async_copy(tb.at[xpr.at[c]], p3, sg),
                pltpu.async_copy(ta.at[xns.at[c]], n1, sg),
                pltpu.async_copy(ta.at[xno.at[c]], n2, sg),
                pltpu.async_copy(tb.at[xnr.at[c]], n3, sg),
                pltpu.async_copy(ta.at[xin.at[c]], bA, sg),
                pltpu.async_copy(tb.at[xin.at[c]], bB, sg),
            ]

        def consume(c):
            p1, p2, p3, n1, n2, n3, bA, bB, _, sw = sets[c % 2]
            for cp in gathers[c % 2]:
                cp.wait()

            def row_body(r, carry):
                accp = jnp.zeros((LANES,), f32)
                accn = jnp.zeros((LANES,), f32)
                for k in range(D // LANES):
                    ksl = pl.ds(k * LANES, LANES)
                    dp = p1[r, ksl] + p3[r, ksl] - p2[r, ksl]
                    accp = accp + dp * dp
                    dn = n1[r, ksl] + n3[r, ksl] - n2[r, ksl]
                    accn = accn + dn * dn
                bB[r, pl.ds(0, LANES)] = accp - accn
                return carry

            lax.fori_loop(0, CHUNK, row_body, 0)
            base = (row0 + c) * CHUNK
            rsl = pl.ds(base, CHUNK)
            writes[c % 2] = [
                pltpu.async_copy(bA, out_a.at[rsl], sw),
                pltpu.async_copy(bB, out_b.at[rsl], sw),
            ]

        for c in range(nch):
            if c >= 2:
                for cp in writes[c % 2]:
                    cp.wait()
            fire(c)
            if c >= 1:
                consume(c - 1)
        consume(nch - 1)
        for s in range(2):
            for cp in writes[s]:
                cp.wait()

    return sc_fn


def _tc_combine_kernel(B, D, D1, D2, RB):
    f32 = jnp.float32
    G = B // RB

    def body(a_ref, b_ref, w1_ref, w2_ref, w_ref, ms_ref,
             loss_ref, se_ref, sg_ref):
        i = pl.program_id(0)
        x = a_ref[:, :D]
        e0 = a_ref[:, D:D + D1]
        dv = b_ref[:, :LANES]
        e1 = b_ref[:, D:D + D2]
        d1 = jnp.dot(x, w1_ref[...], preferred_element_type=f32,
                     precision=lax.Precision.HIGHEST) - e0
        d2 = jnp.dot(x, w2_ref[...], preferred_element_type=f32,
                     precision=lax.Precision.HIGHEST) - e1
        s_emb = (jnp.sum(jnp.sqrt(jnp.sum(d1 * d1, axis=1))) +
                 jnp.sum(jnp.sqrt(jnp.sum(d2 * d2, axis=1))))
        sd = jnp.sum(dv, axis=1)
        hinge = jnp.maximum(0.0, 1.0 + sd)
        wv = w_ref[0, 0, :]
        ms = ms_ref[0, 0]
        s_graph = ms * jnp.sum(hinge) + jnp.sum(wv * hinge)

        @pl.when(i == 0)
        def _():
            se_ref[0, 0] = 0.0
            sg_ref[0, 0] = 0.0

        se_ref[0, 0] += s_emb
        sg_ref[0, 0] += s_graph

        @pl.when(i == pl.num_programs(0) - 1)
        def _():
            wn = jnp.sum(jnp.abs(w1_ref[...])) + jnp.sum(jnp.abs(w2_ref[...]))
            se_t = se_ref[0, 0] + L2_C * wn
            sg_t = sg_ref[0, 0]
            se_ref[0, 0] = se_t
            loss_ref[0, 0] = se_t + (NUM_EMBS / 2.0) * GAMMA_C * sg_t / (ms + ms)

    return pl.pallas_call(
        body,
        grid=(G,),
        in_specs=[
            pl.BlockSpec((RB, PK), lambda i: (i, 0)),
            pl.BlockSpec((RB, PK), lambda i: (i, 0)),
            pl.BlockSpec((D, D1), lambda i: (0, 0)),
            pl.BlockSpec((D, D2), lambda i: (0, 0)),
            pl.BlockSpec((1, 1, RB), lambda i: (i, 0, 0)),
            pl.BlockSpec(memory_space=pltpu.SMEM),
        ],
        out_specs=[
            pl.BlockSpec(memory_space=pltpu.SMEM),
            pl.BlockSpec(memory_space=pltpu.SMEM),
            pl.BlockSpec(memory_space=pltpu.SMEM),
        ],
        out_shape=[
            jax.ShapeDtypeStruct((1, 1), f32),
            jax.ShapeDtypeStruct((1, 1), f32),
            jax.ShapeDtypeStruct((1, 1), f32),
        ],
    )


def kernel(inp, pos_samples, neg_samples, weight, meanscore, metaemb, relemb,
           emb0, emb1, W1, W2):
    B = inp.shape[0]
    D = metaemb.shape[1]
    D1 = emb0.shape[1]
    D2 = emb1.shape[1]
    V1 = metaemb.shape[0]
    i32 = jnp.int32
    f32 = jnp.float32

    ta = jnp.concatenate([metaemb, emb0], axis=1)
    (ta,) = lax.optimization_barrier((ta,))
    tb = jnp.concatenate([relemb, emb1,
                          jnp.zeros((V1, PK - D - D2), f32)], axis=1)
    (tb,) = lax.optimization_barrier((tb,))

    nct = B // CHUNK
    in2 = inp.astype(i32).reshape(nct, CHUNK)
    ps2 = pos_samples[:, 0].astype(i32).reshape(nct, CHUNK)
    pr2 = pos_samples[:, 1].astype(i32).reshape(nct, CHUNK)
    po2 = pos_samples[:, 2].astype(i32).reshape(nct, CHUNK)
    ns2 = neg_samples[:, 0].astype(i32).reshape(nct, CHUNK)
    nr2 = neg_samples[:, 1].astype(i32).reshape(nct, CHUNK)
    no2 = neg_samples[:, 2].astype(i32).reshape(nct, CHUNK)

    sc_fn = _sc_gather_kernel(B, D)
    out_a, out_b = sc_fn(ta, tb, ps2, pr2, po2, ns2, nr2, no2, in2)

    RB = 2048
    tc_fn = _tc_combine_kernel(B, D, D1, D2, RB)
    loss11, se11, sg11 = tc_fn(out_a, out_b, W1, W2,
                               weight.reshape(B // RB, 1, RB),
                               meanscore.reshape(1, 1))
    return (loss11.reshape(1), se11.reshape(()), sg11.reshape(()))


# speedup vs baseline: 1.6271x; 1.3170x over previous
import functools

import jax
import jax.numpy as jnp
from jax import lax
from jax.experimental import pallas as pl
from jax.experimental.pallas import tpu as pltpu
from jax.experimental.pallas import tpu_sc as plsc

GAMMA_C = 1.0
L2_C = 0.0001
NUM_EMBS = 2

NC = 2
NS = 16
LANES = 16
CHUNK = 32
PK = 128


def _sc_gather_kernel(B, D):
    nw = NC * NS
    per_w = B // nw
    nch = per_w // CHUNK
    f32 = jnp.float32
    i32 = jnp.int32
    mesh = plsc.VectorSubcoreMesh(
        core_axis_name="c", subcore_axis_name="s", num_cores=NC, num_subcores=NS
    )

    idx_scratch = [pltpu.VMEM((nch, CHUNK), i32) for _ in range(7)]
    set_scratch = []
    for _ in range(2):
        set_scratch += [pltpu.VMEM((CHUNK, PK), f32) for _ in range(8)]

    @functools.partial(
        pl.kernel,
        out_type=[
            jax.ShapeDtypeStruct((B, PK), f32),
            jax.ShapeDtypeStruct((B, PK), f32),
        ],
        mesh=mesh,
        compiler_params=pltpu.CompilerParams(use_tc_tiling_on_sc=True),
        scratch_types=idx_scratch + set_scratch + [
            pltpu.SemaphoreType.DMA,
            pltpu.SemaphoreType.DMA,
            pltpu.SemaphoreType.DMA,
            pltpu.SemaphoreType.DMA,
        ],
    )
    def sc_fn(ta, tb, ps2, pr2, po2, ns2, nr2, no2, in2,
              out_a, out_b,
              xps, xpr, xpo, xns, xnr, xno, xin,
              p1a, p2a, p3a, n1a, n2a, n3a, bAa, bBa,
              p1b, p2b, p3b, n1b, n2b, n3b, bAb, bBb,
              sg0, sg1, sw0, sw1):
        wid = lax.axis_index("s") * NC + lax.axis_index("c")
        row0 = wid * nch
        for src, dst in ((ps2, xps), (pr2, xpr), (po2, xpo), (ns2, xns),
                         (nr2, xnr), (no2, xno), (in2, xin)):
            pltpu.sync_copy(src.at[pl.ds(row0, nch)], dst)

        sets = (
            (p1a, p2a, p3a, n1a, n2a, n3a, bAa, bBa, sg0, sw0),
            (p1b, p2b, p3b, n1b, n2b, n3b, bAb, bBb, sg1, sw1),
        )
        gathers = [None, None]
        writes = [None, None]

        def fire(c):
            p1, p2, p3, n1, n2, n3, bA, bB, sg, _ = sets[c % 2]
            gathers[c % 2] = [
                pltpu.async_copy(ta.at[xps.at[c]], p1, sg),
                pltpu.async_copy(ta.at[xpo.at[c]], p2, sg),
                pltpu.async_copy(tb.at[xpr.at[c]], p3, sg),
                pltpu.async_copy(ta.at[xns.at[c]], n1, sg),
                pltpu.async_copy(ta.at[xno.at[c]], n2, sg),
                pltpu.async_copy(tb.at[xnr.at[c]], n3, sg),
                pltpu.async_copy(ta.at[xin.at[c]], bA, sg),
                pltpu.async_copy(tb.at[xin.at[c]], bB, sg),
            ]

        def consume(c):
            p1, p2, p3, n1, n2, n3, bA, bB, _, sw = sets[c % 2]
            for cp in gathers[c % 2]:
                cp.wait()

            def row_body(r, carry):
                accp = jnp.zeros((LANES,), f32)
                accn = jnp.zeros((LANES,), f32)
                for k in range(D // LANES):
                    ksl = pl.ds(k * LANES, LANES)
                    dp = p1[r, ksl] + p3[r, ksl] - p2[r, ksl]
                    accp = accp + dp * dp
                    dn = n1[r, ksl] + n3[r, ksl] - n2[r, ksl]
                    accn = accn + dn * dn
                bB[r, pl.ds(0, LANES)] = accp - accn
                return carry

            lax.fori_loop(0, CHUNK, row_body, 0)
            base = (row0 + c) * CHUNK
            rsl = pl.ds(base, CHUNK)
            writes[c % 2] = [
                pltpu.async_copy(bA, out_a.at[rsl], sw),
                pltpu.async_copy(bB, out_b.at[rsl], sw),
            ]

        for c in range(nch):
            if c >= 2:
                for cp in writes[c % 2]:
                    cp.wait()
            fire(c)
            if c >= 1:
                consume(c - 1)
        consume(nch - 1)
        for s in range(2):
            for cp in writes[s]:
                cp.wait()

    return sc_fn


def _tc_pack_kernel(V1, D, D1, D2, CB):
    f32 = jnp.float32
    G = (V1 + CB - 1) // CB

    def body(mt_ref, e0t_ref, rt_ref, e1t_ref, oa_ref, ob_ref):
        oa_ref[:, :D] = jnp.swapaxes(mt_ref[...], 0, 1)
        oa_ref[:, D:D + D1] = jnp.swapaxes(e0t_ref[...], 0, 1)
        ob_ref[:, :D] = jnp.swapaxes(rt_ref[...], 0, 1)
        ob_ref[:, D:D + D2] = jnp.swapaxes(e1t_ref[...], 0, 1)

    return pl.pallas_call(
        body,
        grid=(G,),
        in_specs=[
            pl.BlockSpec((D, CB), lambda i: (0, i)),
            pl.BlockSpec((D1, CB), lambda i: (0, i)),
            pl.BlockSpec((D, CB), lambda i: (0, i)),
            pl.BlockSpec((D2, CB), lambda i: (0, i)),
        ],
        out_specs=[
            pl.BlockSpec((CB, PK), lambda i: (i, 0)),
            pl.BlockSpec((CB, PK), lambda i: (i, 0)),
        ],
        out_shape=[
            jax.ShapeDtypeStruct((V1, PK), f32),
            jax.ShapeDtypeStruct((V1, PK), f32),
        ],
    )


def _tc_combine_kernel(B, D, D1, D2, RB):
    f32 = jnp.float32
    G = B // RB

    def body(a_ref, b_ref, w1_ref, w2_ref, w_ref, ms_ref,
             loss_ref, se_ref, sg_ref):
        i = pl.program_id(0)
        x = a_ref[:, :D]
        e0 = a_ref[:, D:D + D1]
        dv = b_ref[:, :LANES]
        e1 = b_ref[:, D:D + D2]
        d1 = jnp.dot(x, w1_ref[...], preferred_element_type=f32,
                     precision=lax.Precision.HIGHEST) - e0
        d2 = jnp.dot(x, w2_ref[...], preferred_element_type=f32,
                     precision=lax.Precision.HIGHEST) - e1
        s_emb = (jnp.sum(jnp.sqrt(jnp.sum(d1 * d1, axis=1))) +
                 jnp.sum(jnp.sqrt(jnp.sum(d2 * d2, axis=1))))
        sd = jnp.sum(dv, axis=1)
        hinge = jnp.maximum(0.0, 1.0 + sd)
        wv = w_ref[0, 0, :]
        ms = ms_ref[0, 0]
        s_graph = ms * jnp.sum(hinge) + jnp.sum(wv * hinge)

        @pl.when(i == 0)
        def _():
            se_ref[0, 0] = 0.0
            sg_ref[0, 0] = 0.0

        se_ref[0, 0] += s_emb
        sg_ref[0, 0] += s_graph

        @pl.when(i == pl.num_programs(0) - 1)
        def _():
            wn = jnp.sum(jnp.abs(w1_ref[...])) + jnp.sum(jnp.abs(w2_ref[...]))
            se_t = se_ref[0, 0] + L2_C * wn
            sg_t = sg_ref[0, 0]
            se_ref[0, 0] = se_t
            loss_ref[0, 0] = se_t + (NUM_EMBS / 2.0) * GAMMA_C * sg_t / (ms + ms)

    return pl.pallas_call(
        body,
        grid=(G,),
        in_specs=[
            pl.BlockSpec((RB, PK), lambda i: (i, 0)),
            pl.BlockSpec((RB, PK), lambda i: (i, 0)),
            pl.BlockSpec((D, D1), lambda i: (0, 0)),
            pl.BlockSpec((D, D2), lambda i: (0, 0)),
            pl.BlockSpec((1, 1, RB), lambda i: (i, 0, 0)),
            pl.BlockSpec(memory_space=pltpu.SMEM),
        ],
        out_specs=[
            pl.BlockSpec(memory_space=pltpu.SMEM),
            pl.BlockSpec(memory_space=pltpu.SMEM),
            pl.BlockSpec(memory_space=pltpu.SMEM),
        ],
        out_shape=[
            jax.ShapeDtypeStruct((1, 1), f32),
            jax.ShapeDtypeStruct((1, 1), f32),
            jax.ShapeDtypeStruct((1, 1), f32),
        ],
    )


def kernel(inp, pos_samples, neg_samples, weight, meanscore, metaemb, relemb,
           emb0, emb1, W1, W2):
    B = inp.shape[0]
    D = metaemb.shape[1]
    D1 = emb0.shape[1]
    D2 = emb1.shape[1]
    V1 = metaemb.shape[0]
    i32 = jnp.int32
    f32 = jnp.float32

    pack_fn = _tc_pack_kernel(V1, D, D1, D2, 2048)
    ta, tb = pack_fn(metaemb.T, emb0.T, relemb.T, emb1.T)

    nct = B // CHUNK
    in2 = inp.astype(i32).reshape(nct, CHUNK)
    ps2 = pos_samples[:, 0].astype(i32).reshape(nct, CHUNK)
    pr2 = pos_samples[:, 1].astype(i32).reshape(nct, CHUNK)
    po2 = pos_samples[:, 2].astype(i32).reshape(nct, CHUNK)
    ns2 = neg_samples[:, 0].astype(i32).reshape(nct, CHUNK)
    nr2 = neg_samples[:, 1].astype(i32).reshape(nct, CHUNK)
    no2 = neg_samples[:, 2].astype(i32).reshape(nct, CHUNK)

    sc_fn = _sc_gather_kernel(B, D)
    out_a, out_b = sc_fn(ta, tb, ps2, pr2, po2, ns2, nr2, no2, in2)

    RB = 2048
    tc_fn = _tc_combine_kernel(B, D, D1, D2, RB)
    loss11, se11, sg11 = tc_fn(out_a, out_b, W1, W2,
                               weight.reshape(B // RB, 1, RB),
                               meanscore.reshape(1, 1))
    return (loss11.reshape(1), se11.reshape(()), sg11.reshape(()))


# speedup vs baseline: 1.7577x; 1.0802x over previous
import functools

import jax
import jax.numpy as jnp
from jax import lax
from jax.experimental import pallas as pl
from jax.experimental.pallas import tpu as pltpu
from jax.experimental.pallas import tpu_sc as plsc

GAMMA_C = 1.0
L2_C = 0.0001
NUM_EMBS = 2

NC = 2
NS = 16
LANES = 16
CHUNK = 32
PK = 128


def _sc_gather_kernel(B, D):
    nw = NC * NS
    per_w = B // nw
    nch = per_w // CHUNK
    f32 = jnp.float32
    i32 = jnp.int32
    mesh = plsc.VectorSubcoreMesh(
        core_axis_name="c", subcore_axis_name="s", num_cores=NC, num_subcores=NS
    )

    idx_scratch = [pltpu.VMEM((nch, CHUNK), i32) for _ in range(7)]
    set_scratch = []
    for _ in range(2):
        set_scratch += [pltpu.VMEM((CHUNK, PK), f32) for _ in range(8)]

    @functools.partial(
        pl.kernel,
        out_type=[
            jax.ShapeDtypeStruct((B, PK), f32),
            jax.ShapeDtypeStruct((B, PK), f32),
        ],
        mesh=mesh,
        compiler_params=pltpu.CompilerParams(use_tc_tiling_on_sc=True),
        scratch_types=idx_scratch + set_scratch + [
            pltpu.SemaphoreType.DMA,
            pltpu.SemaphoreType.DMA,
            pltpu.SemaphoreType.DMA,
            pltpu.SemaphoreType.DMA,
        ],
    )
    def sc_fn(ta, tb, ps2, pr2, po2, ns2, nr2, no2, in2,
              out_a, out_b,
              xps, xpr, xpo, xns, xnr, xno, xin,
              p1a, p2a, p3a, n1a, n2a, n3a, bAa, bBa,
              p1b, p2b, p3b, n1b, n2b, n3b, bAb, bBb,
              sg0, sg1, sw0, sw1):
        wid = lax.axis_index("s") * NC + lax.axis_index("c")
        row0 = wid * nch
        for src, dst in ((ps2, xps), (pr2, xpr), (po2, xpo), (ns2, xns),
                         (nr2, xnr), (no2, xno), (in2, xin)):
            pltpu.sync_copy(src.at[pl.ds(row0, nch)], dst)

        sets = (
            (p1a, p2a, p3a, n1a, n2a, n3a, bAa, bBa, sg0, sw0),
            (p1b, p2b, p3b, n1b, n2b, n3b, bAb, bBb, sg1, sw1),
        )
        gathers = [None, None]
        writes = [None, None]

        def fire(c):
            p1, p2, p3, n1, n2, n3, bA, bB, sg, _ = sets[c % 2]
            gathers[c % 2] = [
                pltpu.async_copy(ta.at[xps.at[c]], p1, sg),
                pltpu.async_copy(ta.at[xpo.at[c]], p2, sg),
                pltpu.async_copy(tb.at[xpr.at[c]], p3, sg),
                pltpu.async_copy(ta.at[xns.at[c]], n1, sg),
                pltpu.async_copy(ta.at[xno.at[c]], n2, sg),
                pltpu.async_copy(tb.at[xnr.at[c]], n3, sg),
                pltpu.async_copy(ta.at[xin.at[c]], bA, sg),
                pltpu.async_copy(tb.at[xin.at[c]], bB, sg),
            ]

        def consume(c):
            p1, p2, p3, n1, n2, n3, bA, bB, _, sw = sets[c % 2]
            for cp in gathers[c % 2]:
                cp.wait()

            def row_body(r, carry):
                accp = jnp.zeros((LANES,), f32)
                accn = jnp.zeros((LANES,), f32)
                for k in range(D // LANES):
                    ksl = pl.ds(k * LANES, LANES)
                    dp = p1[r, ksl] + p3[r, ksl] - p2[r, ksl]
                    accp = accp + dp * dp
                    dn = n1[r, ksl] + n3[r, ksl] - n2[r, ksl]
                    accn = accn + dn * dn
                bB[r, pl.ds(0, LANES)] = accp - accn
                return carry

            lax.fori_loop(0, CHUNK, row_body, 0)
            base = (row0 + c) * CHUNK
            rsl = pl.ds(base, CHUNK)
            writes[c % 2] = [
                pltpu.async_copy(bA, out_a.at[rsl], sw),
                pltpu.async_copy(bB, out_b.at[rsl], sw),
            ]

        for c in range(nch):
            if c >= 2:
                for cp in writes[c % 2]:
                    cp.wait()
            fire(c)
            if c >= 1:
                consume(c - 1)
        consume(nch - 1)
        for s in range(2):
            for cp in writes[s]:
                cp.wait()

    return sc_fn


def _tc_pack_kernel(V1, D, D1, D2, CB):
    f32 = jnp.float32
    G = (V1 + CB - 1) // CB

    def body(mt_ref, e0t_ref, rt_ref, e1t_ref, oa_ref, ob_ref):
        oa_ref[:, :D] = jnp.swapaxes(mt_ref[...], 0, 1)
        oa_ref[:, D:D + D1] = jnp.swapaxes(e0t_ref[...], 0, 1)
        ob_ref[:, :D] = jnp.swapaxes(rt_ref[...], 0, 1)
        ob_ref[:, D:D + D2] = jnp.swapaxes(e1t_ref[...], 0, 1)

    return pl.pallas_call(
        body,
        grid=(G,),
        in_specs=[
            pl.BlockSpec((D, CB), lambda i: (0, i)),
            pl.BlockSpec((D1, CB), lambda i: (0, i)),
            pl.BlockSpec((D, CB), lambda i: (0, i)),
            pl.BlockSpec((D2, CB), lambda i: (0, i)),
        ],
        out_specs=[
            pl.BlockSpec((CB, PK), lambda i: (i, 0)),
            pl.BlockSpec((CB, PK), lambda i: (i, 0)),
        ],
        out_shape=[
            jax.ShapeDtypeStruct((V1, PK), f32),
            jax.ShapeDtypeStruct((V1, PK), f32),
        ],
    )


def _tc_combine_kernel(B, D, D1, D2, RB):
    f32 = jnp.float32
    G = B // RB

    def body(a_ref, b_ref, w1_ref, w2_ref, w_ref, ms_ref,
             loss_ref, se_ref, sg_ref):
        i = pl.program_id(0)
        x = a_ref[:, :D]
        e0 = a_ref[:, D:D + D1]
        dv = b_ref[:, :LANES]
        e1 = b_ref[:, D:D + D2]
        d1 = jnp.dot(x, w1_ref[...], preferred_element_type=f32,
                     precision=lax.Precision.DEFAULT) - e0
        d2 = jnp.dot(x, w2_ref[...], preferred_element_type=f32,
                     precision=lax.Precision.DEFAULT) - e1
        s_emb = (jnp.sum(jnp.sqrt(jnp.sum(d1 * d1, axis=1))) +
                 jnp.sum(jnp.sqrt(jnp.sum(d2 * d2, axis=1))))
        sd = jnp.sum(dv, axis=1)
        hinge = jnp.maximum(0.0, 1.0 + sd)
        wv = w_ref[0, 0, :]
        ms = ms_ref[0, 0]
        s_graph = ms * jnp.sum(hinge) + jnp.sum(wv * hinge)

        @pl.when(i == 0)
        def _():
            se_ref[0, 0] = 0.0
            sg_ref[0, 0] = 0.0

        se_ref[0, 0] += s_emb
        sg_ref[0, 0] += s_graph

        @pl.when(i == pl.num_programs(0) - 1)
        def _():
            wn = jnp.sum(jnp.abs(w1_ref[...])) + jnp.sum(jnp.abs(w2_ref[...]))
            se_t = se_ref[0, 0] + L2_C * wn
            sg_t = sg_ref[0, 0]
            se_ref[0, 0] = se_t
            loss_ref[0, 0] = se_t + (NUM_EMBS / 2.0) * GAMMA_C * sg_t / (ms + ms)

    return pl.pallas_call(
        body,
        grid=(G,),
        in_specs=[
            pl.BlockSpec((RB, PK), lambda i: (i, 0)),
            pl.BlockSpec((RB, PK), lambda i: (i, 0)),
            pl.BlockSpec((D, D1), lambda i: (0, 0)),
            pl.BlockSpec((D, D2), lambda i: (0, 0)),
            pl.BlockSpec((1, 1, RB), lambda i: (i, 0, 0)),
            pl.BlockSpec(memory_space=pltpu.SMEM),
        ],
        out_specs=[
            pl.BlockSpec(memory_space=pltpu.SMEM),
            pl.BlockSpec(memory_space=pltpu.SMEM),
            pl.BlockSpec(memory_space=pltpu.SMEM),
        ],
        out_shape=[
            jax.ShapeDtypeStruct((1, 1), f32),
            jax.ShapeDtypeStruct((1, 1), f32),
            jax.ShapeDtypeStruct((1, 1), f32),
        ],
    )


def kernel(inp, pos_samples, neg_samples, weight, meanscore, metaemb, relemb,
           emb0, emb1, W1, W2):
    B = inp.shape[0]
    D = metaemb.shape[1]
    D1 = emb0.shape[1]
    D2 = emb1.shape[1]
    V1 = metaemb.shape[0]
    i32 = jnp.int32
    f32 = jnp.float32

    pack_fn = _tc_pack_kernel(V1, D, D1, D2, 4096)
    ta, tb = pack_fn(metaemb.T, emb0.T, relemb.T, emb1.T)

    nct = B // CHUNK
    in2 = inp.astype(i32).reshape(nct, CHUNK)
    ps2 = pos_samples[:, 0].astype(i32).reshape(nct, CHUNK)
    pr2 = pos_samples[:, 1].astype(i32).reshape(nct, CHUNK)
    po2 = pos_samples[:, 2].astype(i32).reshape(nct, CHUNK)
    ns2 = neg_samples[:, 0].astype(i32).reshape(nct, CHUNK)
    nr2 = neg_samples[:, 1].astype(i32).reshape(nct, CHUNK)
    no2 = neg_samples[:, 2].astype(i32).reshape(nct, CHUNK)

    sc_fn = _sc_gather_kernel(B, D)
    out_a, out_b = sc_fn(ta, tb, ps2, pr2, po2, ns2, nr2, no2, in2)

    RB = 4096
    tc_fn = _tc_combine_kernel(B, D, D1, D2, RB)
    loss11, se11, sg11 = tc_fn(out_a, out_b, W1, W2,
                               weight.reshape(B // RB, 1, RB),
                               meanscore.reshape(1, 1))
    return (loss11.reshape(1), se11.reshape(()), sg11.reshape(()))
